# baseline (speedup 1.0000x reference)
import jax
import jax.numpy as jnp
from jax import lax
from jax.experimental import pallas as pl
from jax.experimental.pallas import tpu as pltpu
from jax.experimental.pallas import tpu_sc as plsc

N = 10000
E = 320000
D = 128
G = 64
EPS = 1e-5

NC = 2
NS = 16
L = 16
K = 80

NPAD = 10240
RPT = NPAD // NS
EPW = E // (NC * NS)
NCHUNK = EPW // K

HALF = 5120
NHIGH = N - HALF
AROWS = 6400
ARPT = AROWS // NS
TRASH = HALF
ES = E // NS
NITER = ES // L
KC = 128
MAXCH = 158
CCAP = MAXCH * KC


def _sc_mesh():
    return plsc.VectorSubcoreMesh(core_axis_name="c", subcore_axis_name="s")


def _route_body(src_hbm, dst_hbm, cc_hbm, nch_hbm,
                sv, dv, cc, cbuf):
    cid = lax.axis_index("c")
    sid = lax.axis_index("s")
    pltpu.sync_copy(src_hbm.at[sid], sv)
    pltpu.sync_copy(dst_hbm.at[sid], dv)
    lo = cid.astype(jnp.int32) * HALF

    trash_v = jnp.full((L,), TRASH, jnp.int32)
    zero_v = jnp.zeros((L,), jnp.int32)

    def fill(i, carry):
        v = jnp.where(((i >> 3) & 1) == 0, zero_v, trash_v)
        cc[pl.ds(i * L, L)] = v
        return carry

    lax.fori_loop(0, 2 * CCAP // L, fill, 0)

    def body(i, off):
        vd = dv[pl.ds(i * L, L)]
        vs = sv[pl.ds(i * L, L)]
        vloc = vd - lo
        m = (vloc >= 0) & (vloc < HALF)
        csum = plsc.cumsum(m.astype(jnp.int32))
        pos = csum + (off - 1)
        base = pos + ((pos >> 7) << 7)
        plsc.store_scatter(cc, [base], vs, mask=m)
        plsc.store_scatter(cc, [base + KC], vloc, mask=m)
        return off + jnp.max(csum)

    off = lax.fori_loop(0, NITER, body, jnp.int32(0))

    nch = off // KC + 1
    rep = jnp.full((L,), 0, jnp.int32) + nch
    for t in range(KC // L):
        cbuf[pl.ds(t * L, L)] = rep
    pltpu.sync_copy(cbuf, nch_hbm.at[cid, sid])
    pltpu.sync_copy(cc, cc_hbm.at[cid, sid])


def _route_call(src2, dst2):
    f = pl.kernel(
        _route_body,
        out_type=(jax.ShapeDtypeStruct((NC, NS, 2 * CCAP), jnp.int32),
                  jax.ShapeDtypeStruct((NC, NS, KC), jnp.int32)),
        mesh=_sc_mesh(),
        compiler_params=pltpu.CompilerParams(needs_layout_passes=False),
        scratch_types=[
            pltpu.VMEM((ES,), jnp.int32),
            pltpu.VMEM((ES,), jnp.int32),
            pltpu.VMEM((2 * CCAP,), jnp.int32),
            pltpu.VMEM((KC,), jnp.int32),
        ],
    )
    return f(src2, dst2)


def _agg_body(h_hbm, cc_hbm, nch_hbm, zeros_hbm, p_hbm,
              cb, buf0, buf1, zbuf, cbuf, acc, gsem0, gsem1):
    cid = lax.axis_index("c")
    sid = lax.axis_index("s")
    pltpu.sync_copy(zeros_hbm, zbuf)
    for t in range(ARPT // K):
        pltpu.sync_copy(zbuf, acc.at[pl.ds(sid * ARPT + t * K, K)])
    pltpu.sync_copy(nch_hbm.at[cid, sid], cbuf)
    nch = jnp.max(cbuf[pl.ds(0, L)])
    plsc.subcore_barrier()

    nch2 = (nch + 1) // 2

    def body(g, carry):
        pltpu.sync_copy(cc_hbm.at[cid, sid, pl.ds(2 * g, 2)], cb)
        pltpu.async_copy(h_hbm.at[cb.at[0, 0]], buf0, gsem0)
        pltpu.async_copy(h_hbm.at[cb.at[1, 0]], buf1, gsem1)
        pltpu.make_async_copy(h_hbm.at[cb.at[0, 0]], buf0, gsem0).wait()
        pltpu.sync_copy(buf0, acc.at[cb.at[0, 1]], add=True)
        pltpu.make_async_copy(h_hbm.at[cb.at[1, 0]], buf1, gsem1).wait()
        pltpu.sync_copy(buf1, acc.at[cb.at[1, 1]], add=True)
        return carry

    lax.fori_loop(0, nch2, body, nch2)
    plsc.subcore_barrier()

    pltpu.sync_copy(acc.at[pl.ds(sid * ARPT, ARPT)],
                    p_hbm.at[cid, pl.ds(sid * ARPT, ARPT)])


def _agg_call(hs, cc4, nch, zerosD):
    f = pl.kernel(
        _agg_body,
        out_type=jax.ShapeDtypeStruct((NC, AROWS, D), jnp.float32),
        mesh=_sc_mesh(),
        compiler_params=pltpu.CompilerParams(needs_layout_passes=False),
        scratch_types=[
            pltpu.VMEM((2, 2, KC), jnp.int32),
            pltpu.VMEM((KC, D), jnp.float32),
            pltpu.VMEM((KC, D), jnp.float32),
            pltpu.VMEM((K, D), jnp.float32),
            pltpu.VMEM((KC,), jnp.int32),
            pltpu.VMEM_SHARED((AROWS, D), jnp.float32),
            pltpu.SemaphoreType.DMA,
            pltpu.SemaphoreType.DMA,
        ],
    )
    return f(hs, cc4, nch, zerosD)


def _deg_body(cc_hbm, nch_hbm, zeros_hbm, ones_hbm, dd_hbm,
              cb, buf0, zbuf, cbuf, acc):
    cid = lax.axis_index("c")
    sid = lax.axis_index("s")
    pltpu.sync_copy(zeros_hbm, zbuf)
    for t in range(ARPT // K):
        pltpu.sync_copy(zbuf, acc.at[pl.ds(sid * ARPT + t * K, K)])
    pltpu.sync_copy(ones_hbm, buf0)
    pltpu.sync_copy(nch_hbm.at[cid, sid], cbuf)
    nch = jnp.max(cbuf[pl.ds(0, L)])
    plsc.subcore_barrier()

    nch2 = (nch + 1) // 2

    def body(g, carry):
        pltpu.sync_copy(cc_hbm.at[cid, sid, pl.ds(2 * g, 2)], cb)
        pltpu.sync_copy(buf0, acc.at[cb.at[0, 1]], add=True)
        pltpu.sync_copy(buf0, acc.at[cb.at[1, 1]], add=True)
        return carry

    lax.fori_loop(0, nch2, body, nch2)
    plsc.subcore_barrier()

    pltpu.sync_copy(acc.at[pl.ds(sid * ARPT, ARPT)],
                    dd_hbm.at[cid, pl.ds(sid * ARPT, ARPT)])


def _deg_call(cc4, nch, zerosD, onesD):
    f = pl.kernel(
        _deg_body,
        out_type=jax.ShapeDtypeStruct((NC, AROWS, D), jnp.float32),
        mesh=_sc_mesh(),
        compiler_params=pltpu.CompilerParams(needs_layout_passes=False),
        scratch_types=[
            pltpu.VMEM((2, 2, KC), jnp.int32),
            pltpu.VMEM((KC, D), jnp.float32),
            pltpu.VMEM((K, D), jnp.float32),
            pltpu.VMEM((KC,), jnp.int32),
            pltpu.VMEM_SHARED((AROWS, D), jnp.float32),
        ],
    )
    return f(cc4, nch, zerosD, onesD)


def _t1_body(dd_ref, x_ref, w1_ref, dinv_ref, h1s_ref):
    dd = dd_ref[...]
    deg = jnp.concatenate([dd[0, :HALF, 0:1], dd[1, :NHIGH, 0:1]],
                          axis=0) + 1.0
    dinvb = jnp.broadcast_to(lax.rsqrt(deg), (N, D))
    dinv_ref[...] = dinvb
    h1s_ref[...] = jnp.dot(x_ref[...], w1_ref[...],
                           preferred_element_type=jnp.float32) * dinvb


def _t1_call(dd, x, W1):
    return pl.pallas_call(
        _t1_body,
        out_shape=(jax.ShapeDtypeStruct((N, D), jnp.float32),
                   jax.ShapeDtypeStruct((N, D), jnp.float32)),
    )(dd, x, W1)


def _ln(conv, g, be):
    mu = jnp.mean(conv, axis=1, keepdims=True)
    var = jnp.mean((conv - mu) ** 2, axis=1, keepdims=True)
    return (conv - mu) * lax.rsqrt(var + EPS) * g + be


def _t2_body(p_ref, hs_ref, hprev_ref, dinv_ref,
             b_ref, g_ref, be_ref, wn_ref, hnext_ref, hnexts_ref):
    p = p_ref[...]
    agg = jnp.concatenate([p[0, :HALF], p[1, :NHIGH]], axis=0)
    conv = dinv_ref[...] * (agg + hs_ref[...]) + b_ref[...]
    act = jnp.maximum(_ln(conv, g_ref[...], be_ref[...]), 0.0) + hprev_ref[...]
    hnext_ref[...] = act
    hnexts_ref[...] = jnp.dot(act, wn_ref[...],
                              preferred_element_type=jnp.float32) * dinv_ref[...]


def _t2_call(p, hs, hprev, dinv, b, g, be, Wn):
    return pl.pallas_call(
        _t2_body,
        out_shape=(jax.ShapeDtypeStruct((N, D), jnp.float32),
                   jax.ShapeDtypeStruct((N, D), jnp.float32)),
    )(p, hs, hprev, dinv, b, g, be, Wn)


def _t3_body(p_ref, hs_ref, dinv_ref, b_ref, g_ref, be_ref,
             batch_ref, out_ref):
    p = p_ref[...]
    agg = jnp.concatenate([p[0, :HALF], p[1, :NHIGH]], axis=0)
    conv = dinv_ref[...] * (agg + hs_ref[...]) + b_ref[...]
    ln = _ln(conv, g_ref[...], be_ref[...])
    onehot = (batch_ref[...] ==
              lax.broadcasted_iota(jnp.int32, (N, G), 1)).astype(jnp.float32)
    sums = lax.dot_general(onehot, ln, (((0,), (0,)), ((), ())),
                           preferred_element_type=jnp.float32)
    counts = lax.dot_general(onehot, jnp.ones((N, 1), jnp.float32),
                             (((0,), (0,)), ((), ())),
                             preferred_element_type=jnp.float32)
    out_ref[...] = sums / jnp.maximum(counts, 1.0)


def _t3_call(p, hs, dinv, b, g, be, batch2d):
    return pl.pallas_call(
        _t3_body,
        out_shape=jax.ShapeDtypeStruct((G, D), jnp.float32),
    )(p, hs, dinv, b, g, be, batch2d)


def kernel(x, edge_index, batch, W1, b1, g1, be1, W2, b2, g2, be2,
           W3, b3, g3, be3):
    src1 = edge_index[0]
    dst1 = edge_index[1]
    zerosD = jnp.zeros((K, D), jnp.float32)

    cc, nch = _route_call(src1.reshape(NS, ES), dst1.reshape(NS, ES))
    cc4 = cc.reshape(NC, NS, MAXCH, 2, KC)
    dd = _deg_call(cc4, nch, zerosD, jnp.ones((KC, D), jnp.float32))
    dinvb, hs = _t1_call(dd, x, W1)

    h = x
    for (b, g, be, Wn) in ((b1, g1, be1, W2), (b2, g2, be2, W3)):
        p = _agg_call(hs, cc4, nch, zerosD)
        h, hs = _t2_call(p, hs, h, dinvb,
                         b.reshape(1, D), g.reshape(1, D), be.reshape(1, D), Wn)
    p = _agg_call(hs, cc4, nch, zerosD)
    return _t3_call(p, hs, dinvb, b3.reshape(1, D), g3.reshape(1, D),
                    be3.reshape(1, D), batch.reshape(N, 1))


# speedup vs baseline: 1.1370x; 1.1370x over previous
import jax
import jax.numpy as jnp
from jax import lax
from jax.experimental import pallas as pl
from jax.experimental.pallas import tpu as pltpu
from jax.experimental.pallas import tpu_sc as plsc

N = 10000
E = 320000
D = 128
G = 64
EPS = 1e-5

NC = 2
NS = 16
L = 16
K = 80

NPAD = 10240
RPT = NPAD // NS
EPW = E // (NC * NS)
NCHUNK = EPW // K

HALF = 5120
NHIGH = N - HALF
AROWS = 6400
ARPT = AROWS // NS
TRASH = HALF
ES = E // NS
NITER = ES // L
KC = 128
MAXCH = 158
CCAP = MAXCH * KC


def _sc_mesh():
    return plsc.VectorSubcoreMesh(core_axis_name="c", subcore_axis_name="s")


def _route_body(src_hbm, dst_hbm, cc_hbm, nch_hbm,
                sv, dv, cc, cbuf):
    cid = lax.axis_index("c")
    sid = lax.axis_index("s")
    pltpu.sync_copy(src_hbm.at[sid], sv)
    pltpu.sync_copy(dst_hbm.at[sid], dv)
    lo = cid.astype(jnp.int32) * HALF

    trash_v = jnp.full((L,), TRASH, jnp.int32)
    zero_v = jnp.zeros((L,), jnp.int32)

    def fill(i, carry):
        v = jnp.where(((i >> 3) & 1) == 0, zero_v, trash_v)
        cc[pl.ds(i * L, L)] = v
        return carry

    lax.fori_loop(0, 2 * CCAP // L, fill, 0)

    def body(i, off):
        vd = dv[pl.ds(i * L, L)]
        vs = sv[pl.ds(i * L, L)]
        vloc = vd - lo
        m = (vloc >= 0) & (vloc < HALF)
        csum = plsc.cumsum(m.astype(jnp.int32))
        pos = csum + (off - 1)
        base = pos + ((pos >> 7) << 7)
        plsc.store_scatter(cc, [base], vs, mask=m)
        plsc.store_scatter(cc, [base + KC], vloc, mask=m)
        return off + jnp.max(csum)

    off = lax.fori_loop(0, NITER, body, jnp.int32(0))

    nch = off // KC + 1
    rep = jnp.full((L,), 0, jnp.int32) + nch
    for t in range(KC // L):
        cbuf[pl.ds(t * L, L)] = rep
    pltpu.sync_copy(cbuf, nch_hbm.at[cid, sid])
    pltpu.sync_copy(cc, cc_hbm.at[cid, sid])


def _route_call(src2, dst2):
    f = pl.kernel(
        _route_body,
        out_type=(jax.ShapeDtypeStruct((NC, NS, 2 * CCAP), jnp.int32),
                  jax.ShapeDtypeStruct((NC, NS, KC), jnp.int32)),
        mesh=_sc_mesh(),
        compiler_params=pltpu.CompilerParams(needs_layout_passes=False),
        scratch_types=[
            pltpu.VMEM((ES,), jnp.int32),
            pltpu.VMEM((ES,), jnp.int32),
            pltpu.VMEM((2 * CCAP,), jnp.int32),
            pltpu.VMEM((KC,), jnp.int32),
        ],
    )
    return f(src2, dst2)


def _agg_body(h_hbm, cc_hbm, nch_hbm, zeros_hbm, p_hbm,
              cb, buf0, zbuf, cbuf, acc, gsem0):
    cid = lax.axis_index("c")
    sid = lax.axis_index("s")
    pltpu.sync_copy(zeros_hbm, zbuf)
    for t in range(ARPT // K):
        pltpu.sync_copy(zbuf, acc.at[pl.ds(sid * ARPT + t * K, K)])
    pltpu.sync_copy(nch_hbm.at[cid, sid], cbuf)
    nch = jnp.max(cbuf[pl.ds(0, L)])
    plsc.subcore_barrier()

    def body(j, carry):
        pltpu.sync_copy(cc_hbm.at[cid, sid, j], cb)
        pltpu.async_copy(h_hbm.at[cb.at[0]], buf0, gsem0)
        pltpu.make_async_copy(h_hbm.at[cb.at[0]], buf0, gsem0).wait()
        pltpu.sync_copy(buf0, acc.at[cb.at[1]], add=True)
        return carry

    lax.fori_loop(0, nch, body, nch)
    plsc.subcore_barrier()

    pltpu.sync_copy(acc.at[pl.ds(sid * ARPT, ARPT)],
                    p_hbm.at[cid, pl.ds(sid * ARPT, ARPT)])


def _agg_call(hs, cc4, nch, zerosD):
    f = pl.kernel(
        _agg_body,
        out_type=jax.ShapeDtypeStruct((NC, AROWS, D), jnp.float32),
        mesh=_sc_mesh(),
        compiler_params=pltpu.CompilerParams(needs_layout_passes=False),
        scratch_types=[
            pltpu.VMEM((2, KC), jnp.int32),
            pltpu.VMEM((KC, D), jnp.float32),
            pltpu.VMEM((K, D), jnp.float32),
            pltpu.VMEM((KC,), jnp.int32),
            pltpu.VMEM_SHARED((AROWS, D), jnp.float32),
            pltpu.SemaphoreType.DMA,
        ],
    )
    return f(hs, cc4, nch, zerosD)


def _deg_body(cc_hbm, nch_hbm, zeros_hbm, ones_hbm, dd_hbm,
              cb, buf0, zbuf, cbuf, acc):
    cid = lax.axis_index("c")
    sid = lax.axis_index("s")
    pltpu.sync_copy(zeros_hbm, zbuf)
    for t in range(ARPT // K):
        pltpu.sync_copy(zbuf, acc.at[pl.ds(sid * ARPT + t * K, K)])
    pltpu.sync_copy(ones_hbm, buf0)
    pltpu.sync_copy(nch_hbm.at[cid, sid], cbuf)
    nch = jnp.max(cbuf[pl.ds(0, L)])
    plsc.subcore_barrier()

    nch2 = (nch + 1) // 2

    def body(g, carry):
        pltpu.sync_copy(cc_hbm.at[cid, sid, pl.ds(2 * g, 2)], cb)
        pltpu.sync_copy(buf0, acc.at[cb.at[0, 1]], add=True)
        pltpu.sync_copy(buf0, acc.at[cb.at[1, 1]], add=True)
        return carry

    lax.fori_loop(0, nch2, body, nch2)
    plsc.subcore_barrier()

    pltpu.sync_copy(acc.at[pl.ds(sid * ARPT, ARPT)],
                    dd_hbm.at[cid, pl.ds(sid * ARPT, ARPT)])


def _deg_call(cc4, nch, zerosD, onesD):
    f = pl.kernel(
        _deg_body,
        out_type=jax.ShapeDtypeStruct((NC, AROWS, D), jnp.float32),
        mesh=_sc_mesh(),
        compiler_params=pltpu.CompilerParams(needs_layout_passes=False),
        scratch_types=[
            pltpu.VMEM((2, 2, KC), jnp.int32),
            pltpu.VMEM((KC, D), jnp.float32),
            pltpu.VMEM((K, D), jnp.float32),
            pltpu.VMEM((KC,), jnp.int32),
            pltpu.VMEM_SHARED((AROWS, D), jnp.float32),
        ],
    )
    return f(cc4, nch, zerosD, onesD)


def _t1_body(dd_ref, x_ref, w1_ref, dinv_ref, h1s_ref):
    dd = dd_ref[...]
    deg = jnp.concatenate([dd[0, :HALF, 0:1], dd[1, :NHIGH, 0:1]],
                          axis=0) + 1.0
    dinvb = jnp.broadcast_to(lax.rsqrt(deg), (N, D))
    dinv_ref[...] = dinvb
    h1s_ref[...] = jnp.dot(x_ref[...], w1_ref[...],
                           preferred_element_type=jnp.float32) * dinvb


def _t1_call(dd, x, W1):
    return pl.pallas_call(
        _t1_body,
        out_shape=(jax.ShapeDtypeStruct((N, D), jnp.float32),
                   jax.ShapeDtypeStruct((N, D), jnp.float32)),
    )(dd, x, W1)


def _ln(conv, g, be):
    mu = jnp.mean(conv, axis=1, keepdims=True)
    var = jnp.mean((conv - mu) ** 2, axis=1, keepdims=True)
    return (conv - mu) * lax.rsqrt(var + EPS) * g + be


def _t2_body(p_ref, hs_ref, hprev_ref, dinv_ref,
             b_ref, g_ref, be_ref, wn_ref, hnext_ref, hnexts_ref):
    p = p_ref[...]
    agg = jnp.concatenate([p[0, :HALF], p[1, :NHIGH]], axis=0)
    conv = dinv_ref[...] * (agg + hs_ref[...]) + b_ref[...]
    act = jnp.maximum(_ln(conv, g_ref[...], be_ref[...]), 0.0) + hprev_ref[...]
    hnext_ref[...] = act
    hnexts_ref[...] = jnp.dot(act, wn_ref[...],
                              preferred_element_type=jnp.float32) * dinv_ref[...]


def _t2_call(p, hs, hprev, dinv, b, g, be, Wn):
    return pl.pallas_call(
        _t2_body,
        out_shape=(jax.ShapeDtypeStruct((N, D), jnp.float32),
                   jax.ShapeDtypeStruct((N, D), jnp.float32)),
    )(p, hs, hprev, dinv, b, g, be, Wn)


def _t3_body(p_ref, hs_ref, dinv_ref, b_ref, g_ref, be_ref,
             batch_ref, out_ref):
    p = p_ref[...]
    agg = jnp.concatenate([p[0, :HALF], p[1, :NHIGH]], axis=0)
    conv = dinv_ref[...] * (agg + hs_ref[...]) + b_ref[...]
    ln = _ln(conv, g_ref[...], be_ref[...])
    onehot = (batch_ref[...] ==
              lax.broadcasted_iota(jnp.int32, (N, G), 1)).astype(jnp.float32)
    sums = lax.dot_general(onehot, ln, (((0,), (0,)), ((), ())),
                           preferred_element_type=jnp.float32)
    counts = lax.dot_general(onehot, jnp.ones((N, 1), jnp.float32),
                             (((0,), (0,)), ((), ())),
                             preferred_element_type=jnp.float32)
    out_ref[...] = sums / jnp.maximum(counts, 1.0)


def _t3_call(p, hs, dinv, b, g, be, batch2d):
    return pl.pallas_call(
        _t3_body,
        out_shape=jax.ShapeDtypeStruct((G, D), jnp.float32),
    )(p, hs, dinv, b, g, be, batch2d)


def kernel(x, edge_index, batch, W1, b1, g1, be1, W2, b2, g2, be2,
           W3, b3, g3, be3):
    src1 = edge_index[0]
    dst1 = edge_index[1]
    zerosD = jnp.zeros((K, D), jnp.float32)

    cc, nch = _route_call(src1.reshape(NS, ES), dst1.reshape(NS, ES))
    cc4 = cc.reshape(NC, NS, MAXCH, 2, KC)
    dd = _deg_call(cc4, nch, zerosD, jnp.ones((KC, D), jnp.float32))
    dinvb, hs = _t1_call(dd, x, W1)

    h = x
    for (b, g, be, Wn) in ((b1, g1, be1, W2), (b2, g2, be2, W3)):
        p = _agg_call(hs, cc4, nch, zerosD)
        h, hs = _t2_call(p, hs, h, dinvb,
                         b.reshape(1, D), g.reshape(1, D), be.reshape(1, D), Wn)
    p = _agg_call(hs, cc4, nch, zerosD)
    return _t3_call(p, hs, dinvb, b3.reshape(1, D), g3.reshape(1, D),
                    be3.reshape(1, D), batch.reshape(N, 1))
